# Initial kernel scaffold; baseline (speedup 1.0000x reference)
import jax
import jax.numpy as jnp
from jax.experimental import pallas as pl


def kernel(hidden_states, weight):
    raise NotImplementedError("write your pallas kernel here")


# speedup vs baseline: 1.6805x; 1.6805x over previous
import jax
import jax.numpy as jnp
from jax.experimental import pallas as pl
from jax.experimental.pallas import tpu as pltpu

HIDDEN = 2048
EXPERTS = 64
TOPK = 8
BLOCK_R = 256
ALPHA = 0.01


def _gate_kernel(seq_len, bsz, hs_ref, wt_ref, idx_ref, w_ref, aux_ref,
                 cnt_ref, ssum_ref):
    step = pl.program_id(0)
    nsteps = pl.num_programs(0)

    @pl.when(step == 0)
    def _init():
        cnt_ref[...] = jnp.zeros_like(cnt_ref)
        ssum_ref[...] = jnp.zeros_like(ssum_ref)
        aux_ref[...] = jnp.zeros_like(aux_ref)

    logits = jnp.dot(hs_ref[...], wt_ref[...],
                     preferred_element_type=jnp.float32)
    m = jnp.max(logits, axis=1, keepdims=True)
    e = jnp.exp(logits - m)
    s = jnp.sum(e, axis=1, keepdims=True)
    scores = e / s

    iota = jax.lax.broadcasted_iota(jnp.int32, scores.shape, 1)
    cur = scores
    vals = []
    ids = []
    for _ in range(TOPK):
        v = jnp.max(cur, axis=1, keepdims=True)
        hit = cur == v
        idx = jnp.min(jnp.where(hit, iota, EXPERTS), axis=1,
                      keepdims=True)
        vals.append(v)
        ids.append(idx)
        cur = jnp.where(iota == idx, -1.0, cur)
    vals8 = jnp.concatenate(vals, axis=1)
    ids8 = jnp.concatenate(ids, axis=1)
    denom = jnp.sum(vals8, axis=1, keepdims=True) + 1e-20
    idx_ref[...] = ids8
    w_ref[...] = vals8 / denom

    sel = (cur < 0.0).astype(jnp.float32)
    counts = jnp.sum(sel, axis=0, keepdims=True)
    sums = jnp.sum(scores, axis=0, keepdims=True)
    b = step // (seq_len // BLOCK_R)
    bio = jax.lax.broadcasted_iota(jnp.int32, (bsz, 1), 0)
    onehot = (bio == b).astype(jnp.float32)
    cnt_ref[...] += onehot * counts
    ssum_ref[...] += onehot * sums

    @pl.when(step == nsteps - 1)
    def _fin():
        ce = cnt_ref[...] * (EXPERTS / (seq_len * TOPK))
        mean_s = ssum_ref[...] * (1.0 / seq_len)
        aux_ref[...] = jnp.sum(ce * mean_s, axis=(0, 1),
                               keepdims=True) * (ALPHA / bsz)


def kernel(hidden_states, weight):
    bsz, seq_len, h = hidden_states.shape
    hs = hidden_states.reshape(bsz * seq_len, h)
    wt = weight.T
    n = bsz * seq_len
    grid = n // BLOCK_R

    import functools
    body = functools.partial(_gate_kernel, seq_len, bsz)
    idx, w, aux = pl.pallas_call(
        body,
        grid=(grid,),
        in_specs=[
            pl.BlockSpec((BLOCK_R, h), lambda i: (i, 0)),
            pl.BlockSpec((h, EXPERTS), lambda i: (0, 0)),
        ],
        out_specs=[
            pl.BlockSpec((BLOCK_R, TOPK), lambda i: (i, 0)),
            pl.BlockSpec((BLOCK_R, TOPK), lambda i: (i, 0)),
            pl.BlockSpec((1, 1), lambda i: (0, 0)),
        ],
        out_shape=[
            jax.ShapeDtypeStruct((n, TOPK), jnp.int32),
            jax.ShapeDtypeStruct((n, TOPK), jnp.float32),
            jax.ShapeDtypeStruct((1, 1), jnp.float32),
        ],
        scratch_shapes=[
            pltpu.VMEM((bsz, EXPERTS), jnp.float32),
            pltpu.VMEM((bsz, EXPERTS), jnp.float32),
        ],
        compiler_params=pltpu.CompilerParams(
            dimension_semantics=("arbitrary",)),
    )(hs, wt)
    return idx, w, aux[0, 0]


# speedup vs baseline: 2.8693x; 1.7074x over previous
import functools

import jax
import jax.numpy as jnp
from jax.experimental import pallas as pl
from jax.experimental.pallas import tpu as pltpu

HIDDEN = 2048
EXPERTS = 64
TOPK = 8
BLOCK_R = 256
ALPHA = 0.01


def _gate_kernel(seq_len, bsz, hs_ref, wt_ref, idx_ref, w_ref, aux_ref,
                 cnt_ref, ssum_ref):
    step = pl.program_id(0)
    nsteps = pl.num_programs(0)

    @pl.when(step == 0)
    def _init():
        cnt_ref[...] = jnp.zeros_like(cnt_ref)
        ssum_ref[...] = jnp.zeros_like(ssum_ref)
        aux_ref[...] = jnp.zeros_like(aux_ref)

    logits = jnp.dot(hs_ref[...], wt_ref[...],
                     preferred_element_type=jnp.float32)
    lt = logits.T
    m = jnp.max(lt, axis=0, keepdims=True)
    e = jnp.exp(lt - m)
    s = jnp.sum(e, axis=0, keepdims=True)
    scores = e / s

    iota = jax.lax.broadcasted_iota(jnp.int32, scores.shape, 0)
    cur = scores
    vals = []
    ids = []
    for _ in range(TOPK):
        v = jnp.max(cur, axis=0, keepdims=True)
        hit = cur == v
        idx = jnp.min(jnp.where(hit, iota, EXPERTS), axis=0,
                      keepdims=True)
        vals.append(v)
        ids.append(idx)
        cur = jnp.where(iota == idx, -1.0, cur)
    vals8 = jnp.concatenate(vals, axis=0)
    ids8 = jnp.concatenate(ids, axis=0)
    denom = jnp.sum(vals8, axis=0, keepdims=True) + 1e-20
    idx_ref[...] = ids8.T
    w_ref[...] = (vals8 / denom).T

    sel = (cur < 0.0).astype(jnp.float32)
    counts = jnp.sum(sel, axis=1, keepdims=True)
    sums = jnp.sum(scores, axis=1, keepdims=True)
    b = step // (seq_len // BLOCK_R)
    bio = jax.lax.broadcasted_iota(jnp.int32, (EXPERTS, bsz), 1)
    onehot = (bio == b).astype(jnp.float32)
    cnt_ref[...] += onehot * counts
    ssum_ref[...] += onehot * sums

    @pl.when(step == nsteps - 1)
    def _fin():
        ce = cnt_ref[...] * (EXPERTS / (seq_len * TOPK))
        mean_s = ssum_ref[...] * (1.0 / seq_len)
        aux_ref[...] = jnp.sum(ce * mean_s, axis=(0, 1),
                               keepdims=True) * (ALPHA / bsz)


def kernel(hidden_states, weight):
    bsz, seq_len, h = hidden_states.shape
    hs = hidden_states.reshape(bsz * seq_len, h)
    wt = weight.T
    n = bsz * seq_len
    grid = n // BLOCK_R

    body = functools.partial(_gate_kernel, seq_len, bsz)
    idx, w, aux = pl.pallas_call(
        body,
        grid=(grid,),
        in_specs=[
            pl.BlockSpec((BLOCK_R, h), lambda i: (i, 0)),
            pl.BlockSpec((h, EXPERTS), lambda i: (0, 0)),
        ],
        out_specs=[
            pl.BlockSpec((BLOCK_R, TOPK), lambda i: (i, 0)),
            pl.BlockSpec((BLOCK_R, TOPK), lambda i: (i, 0)),
            pl.BlockSpec((1, 1), lambda i: (0, 0)),
        ],
        out_shape=[
            jax.ShapeDtypeStruct((n, TOPK), jnp.int32),
            jax.ShapeDtypeStruct((n, TOPK), jnp.float32),
            jax.ShapeDtypeStruct((1, 1), jnp.float32),
        ],
        scratch_shapes=[
            pltpu.VMEM((EXPERTS, bsz), jnp.float32),
            pltpu.VMEM((EXPERTS, bsz), jnp.float32),
        ],
        compiler_params=pltpu.CompilerParams(
            dimension_semantics=("arbitrary",)),
    )(hs, wt)
    return idx, w, aux[0, 0]


# speedup vs baseline: 3.0810x; 1.0738x over previous
import functools

import jax
import jax.numpy as jnp
from jax.experimental import pallas as pl
from jax.experimental.pallas import tpu as pltpu

HIDDEN = 2048
EXPERTS = 64
TOPK = 8
BLOCK_R = 256
ALPHA = 0.01


def _gate_kernel(seq_len, bsz, wt_ref, hs_ref, idx_ref, w_ref, aux_ref,
                 cnt_ref, ssum_ref):
    step = pl.program_id(0)
    nsteps = pl.num_programs(0)

    @pl.when(step == 0)
    def _init():
        cnt_ref[...] = jnp.zeros_like(cnt_ref)
        ssum_ref[...] = jnp.zeros_like(ssum_ref)
        aux_ref[...] = jnp.zeros_like(aux_ref)

    lt = jax.lax.dot_general(
        wt_ref[...], hs_ref[...],
        dimension_numbers=(((1,), (1,)), ((), ())),
        preferred_element_type=jnp.float32)
    m = jnp.max(lt, axis=0, keepdims=True)
    e = jnp.exp(lt - m)
    s = jnp.sum(e, axis=0, keepdims=True)
    scores = e / s

    iota = jax.lax.broadcasted_iota(jnp.int32, scores.shape, 0)
    cur = scores
    vals = []
    ids = []
    for _ in range(TOPK):
        v = jnp.max(cur, axis=0, keepdims=True)
        hit = cur == v
        idx = jnp.min(jnp.where(hit, iota, EXPERTS), axis=0,
                      keepdims=True)
        vals.append(v)
        ids.append(idx)
        cur = jnp.where(iota == idx, -1.0, cur)
    vals8 = jnp.concatenate(vals, axis=0)
    ids8 = jnp.concatenate(ids, axis=0)
    denom = jnp.sum(vals8, axis=0, keepdims=True) + 1e-20
    idx_ref[...] = ids8.T
    w_ref[...] = (vals8 / denom).T

    sel = (cur < 0.0).astype(jnp.float32)
    counts = jnp.sum(sel, axis=1, keepdims=True)
    sums = jnp.sum(scores, axis=1, keepdims=True)
    b = step // (seq_len // BLOCK_R)
    bio = jax.lax.broadcasted_iota(jnp.int32, (EXPERTS, bsz), 1)
    onehot = (bio == b).astype(jnp.float32)
    cnt_ref[...] += onehot * counts
    ssum_ref[...] += onehot * sums

    @pl.when(step == nsteps - 1)
    def _fin():
        ce = cnt_ref[...] * (EXPERTS / (seq_len * TOPK))
        mean_s = ssum_ref[...] * (1.0 / seq_len)
        aux_ref[...] = jnp.sum(ce * mean_s, axis=(0, 1),
                               keepdims=True) * (ALPHA / bsz)


def kernel(hidden_states, weight):
    bsz, seq_len, h = hidden_states.shape
    hs = hidden_states.reshape(bsz * seq_len, h)
    n = bsz * seq_len
    grid = n // BLOCK_R

    body = functools.partial(_gate_kernel, seq_len, bsz)
    idx, w, aux = pl.pallas_call(
        body,
        grid=(grid,),
        in_specs=[
            pl.BlockSpec((EXPERTS, h), lambda i: (0, 0)),
            pl.BlockSpec((BLOCK_R, h), lambda i: (i, 0)),
        ],
        out_specs=[
            pl.BlockSpec((BLOCK_R, TOPK), lambda i: (i, 0)),
            pl.BlockSpec((BLOCK_R, TOPK), lambda i: (i, 0)),
            pl.BlockSpec((1, 1), lambda i: (0, 0)),
        ],
        out_shape=[
            jax.ShapeDtypeStruct((n, TOPK), jnp.int32),
            jax.ShapeDtypeStruct((n, TOPK), jnp.float32),
            jax.ShapeDtypeStruct((1, 1), jnp.float32),
        ],
        scratch_shapes=[
            pltpu.VMEM((EXPERTS, bsz), jnp.float32),
            pltpu.VMEM((EXPERTS, bsz), jnp.float32),
        ],
        compiler_params=pltpu.CompilerParams(
            dimension_semantics=("arbitrary",)),
    )(weight, hs)
    return idx, w, aux[0, 0]


# speedup vs baseline: 3.9237x; 1.2735x over previous
import functools

import jax
import jax.numpy as jnp
from jax.experimental import pallas as pl
from jax.experimental.pallas import tpu as pltpu

HIDDEN = 2048
EXPERTS = 64
TOPK = 8
BLOCK_R = 512
ALPHA = 0.01


def _gate_kernel(seq_len, bsz, wt_ref, hs_ref, idx_ref, w_ref, aux_ref,
                 cnt_ref, ssum_ref):
    step = pl.program_id(0)
    nsteps = pl.num_programs(0)

    @pl.when(step == 0)
    def _init():
        cnt_ref[...] = jnp.zeros_like(cnt_ref)
        ssum_ref[...] = jnp.zeros_like(ssum_ref)
        aux_ref[...] = jnp.zeros_like(aux_ref)

    lt = jax.lax.dot_general(
        wt_ref[...], hs_ref[...],
        dimension_numbers=(((1,), (1,)), ((), ())),
        preferred_element_type=jnp.float32)
    m = jnp.max(lt, axis=0, keepdims=True)
    e = jnp.exp(lt - m)
    s = jnp.sum(e, axis=0, keepdims=True)
    scores = e / s

    iota = jax.lax.broadcasted_iota(jnp.int32, scores.shape, 0)
    cur = scores
    vals = []
    ids = []
    for _ in range(TOPK):
        v = jnp.max(cur, axis=0, keepdims=True)
        hit = cur == v
        idx = jnp.min(jnp.where(hit, iota, EXPERTS), axis=0,
                      keepdims=True)
        vals.append(v)
        ids.append(idx)
        cur = jnp.where(iota == idx, -1.0, cur)
    vals8 = jnp.concatenate(vals, axis=0)
    ids8 = jnp.concatenate(ids, axis=0)
    denom = jnp.sum(vals8, axis=0, keepdims=True) + 1e-20
    idx_ref[...] = ids8.T
    w_ref[...] = (vals8 / denom).T

    sel = (cur < 0.0).astype(jnp.float32)
    counts = jnp.sum(sel, axis=1, keepdims=True)
    sums = jnp.sum(scores, axis=1, keepdims=True)
    b = step // (seq_len // BLOCK_R)
    bio = jax.lax.broadcasted_iota(jnp.int32, (EXPERTS, bsz), 1)
    onehot = (bio == b).astype(jnp.float32)
    cnt_ref[...] += onehot * counts
    ssum_ref[...] += onehot * sums

    @pl.when(step == nsteps - 1)
    def _fin():
        ce = cnt_ref[...] * (EXPERTS / (seq_len * TOPK))
        mean_s = ssum_ref[...] * (1.0 / seq_len)
        aux_ref[...] = jnp.sum(ce * mean_s, axis=(0, 1),
                               keepdims=True) * (ALPHA / bsz)


def kernel(hidden_states, weight):
    bsz, seq_len, h = hidden_states.shape
    hs = hidden_states.reshape(bsz * seq_len, h)
    n = bsz * seq_len
    grid = n // BLOCK_R

    body = functools.partial(_gate_kernel, seq_len, bsz)
    idx, w, aux = pl.pallas_call(
        body,
        grid=(grid,),
        in_specs=[
            pl.BlockSpec((EXPERTS, h), lambda i: (0, 0)),
            pl.BlockSpec((BLOCK_R, h), lambda i: (i, 0)),
        ],
        out_specs=[
            pl.BlockSpec((BLOCK_R, TOPK), lambda i: (i, 0)),
            pl.BlockSpec((BLOCK_R, TOPK), lambda i: (i, 0)),
            pl.BlockSpec((1, 1), lambda i: (0, 0)),
        ],
        out_shape=[
            jax.ShapeDtypeStruct((n, TOPK), jnp.int32),
            jax.ShapeDtypeStruct((n, TOPK), jnp.float32),
            jax.ShapeDtypeStruct((1, 1), jnp.float32),
        ],
        scratch_shapes=[
            pltpu.VMEM((EXPERTS, bsz), jnp.float32),
            pltpu.VMEM((EXPERTS, bsz), jnp.float32),
        ],
        compiler_params=pltpu.CompilerParams(
            dimension_semantics=("arbitrary",)),
    )(weight, hs)
    return idx, w, aux[0, 0]


# speedup vs baseline: 4.5352x; 1.1559x over previous
import functools

import jax
import jax.numpy as jnp
from jax.experimental import pallas as pl
from jax.experimental.pallas import tpu as pltpu

HIDDEN = 2048
EXPERTS = 64
TOPK = 8
BLOCK_R = 1024
ALPHA = 0.01


def _gate_kernel(seq_len, bsz, wt_ref, hs_ref, idx_ref, w_ref, aux_ref,
                 cnt_ref, ssum_ref):
    step = pl.program_id(0)
    nsteps = pl.num_programs(0)

    @pl.when(step == 0)
    def _init():
        cnt_ref[...] = jnp.zeros_like(cnt_ref)
        ssum_ref[...] = jnp.zeros_like(ssum_ref)
        aux_ref[...] = jnp.zeros_like(aux_ref)

    lt = jax.lax.dot_general(
        wt_ref[...], hs_ref[...],
        dimension_numbers=(((1,), (1,)), ((), ())),
        preferred_element_type=jnp.float32)
    m = jnp.max(lt, axis=0, keepdims=True)
    e = jnp.exp(lt - m)
    s = jnp.sum(e, axis=0, keepdims=True)
    scores = e / s

    iota = jax.lax.broadcasted_iota(jnp.int32, scores.shape, 0)
    cur = scores
    vals = []
    ids = []
    for _ in range(TOPK):
        v = jnp.max(cur, axis=0, keepdims=True)
        hit = cur == v
        idx = jnp.min(jnp.where(hit, iota, EXPERTS), axis=0,
                      keepdims=True)
        vals.append(v)
        ids.append(idx)
        cur = jnp.where(iota == idx, -1.0, cur)
    vals8 = jnp.concatenate(vals, axis=0)
    ids8 = jnp.concatenate(ids, axis=0)
    denom = jnp.sum(vals8, axis=0, keepdims=True) + 1e-20
    idx_ref[...] = ids8.T
    w_ref[...] = (vals8 / denom).T

    sel = (cur < 0.0).astype(jnp.float32)
    counts = jnp.sum(sel, axis=1, keepdims=True)
    sums = jnp.sum(scores, axis=1, keepdims=True)
    b = step // (seq_len // BLOCK_R)
    bio = jax.lax.broadcasted_iota(jnp.int32, (EXPERTS, bsz), 1)
    onehot = (bio == b).astype(jnp.float32)
    cnt_ref[...] += onehot * counts
    ssum_ref[...] += onehot * sums

    @pl.when(step == nsteps - 1)
    def _fin():
        ce = cnt_ref[...] * (EXPERTS / (seq_len * TOPK))
        mean_s = ssum_ref[...] * (1.0 / seq_len)
        aux_ref[...] = jnp.sum(ce * mean_s, axis=(0, 1),
                               keepdims=True) * (ALPHA / bsz)


def kernel(hidden_states, weight):
    bsz, seq_len, h = hidden_states.shape
    hs = hidden_states.reshape(bsz * seq_len, h)
    n = bsz * seq_len
    grid = n // BLOCK_R

    body = functools.partial(_gate_kernel, seq_len, bsz)
    idx, w, aux = pl.pallas_call(
        body,
        grid=(grid,),
        in_specs=[
            pl.BlockSpec((EXPERTS, h), lambda i: (0, 0)),
            pl.BlockSpec((BLOCK_R, h), lambda i: (i, 0)),
        ],
        out_specs=[
            pl.BlockSpec((BLOCK_R, TOPK), lambda i: (i, 0)),
            pl.BlockSpec((BLOCK_R, TOPK), lambda i: (i, 0)),
            pl.BlockSpec((1, 1), lambda i: (0, 0)),
        ],
        out_shape=[
            jax.ShapeDtypeStruct((n, TOPK), jnp.int32),
            jax.ShapeDtypeStruct((n, TOPK), jnp.float32),
            jax.ShapeDtypeStruct((1, 1), jnp.float32),
        ],
        scratch_shapes=[
            pltpu.VMEM((EXPERTS, bsz), jnp.float32),
            pltpu.VMEM((EXPERTS, bsz), jnp.float32),
        ],
        compiler_params=pltpu.CompilerParams(
            dimension_semantics=("arbitrary",)),
    )(weight, hs)
    return idx, w, aux[0, 0]


# speedup vs baseline: 4.6185x; 1.0184x over previous
import functools

import jax
import jax.numpy as jnp
from jax.experimental import pallas as pl
from jax.experimental.pallas import tpu as pltpu

HIDDEN = 2048
EXPERTS = 64
TOPK = 8
BLOCK_R = 2048
ALPHA = 0.01


def _gate_kernel(seq_len, bsz, wt_ref, hs_ref, idx_ref, w_ref, aux_ref,
                 cnt_ref, ssum_ref):
    step = pl.program_id(0)
    nsteps = pl.num_programs(0)

    @pl.when(step == 0)
    def _init():
        cnt_ref[...] = jnp.zeros_like(cnt_ref)
        ssum_ref[...] = jnp.zeros_like(ssum_ref)
        aux_ref[...] = jnp.zeros_like(aux_ref)

    lt = jax.lax.dot_general(
        wt_ref[...], hs_ref[...],
        dimension_numbers=(((1,), (1,)), ((), ())),
        preferred_element_type=jnp.float32)
    m = jnp.max(lt, axis=0, keepdims=True)
    e = jnp.exp(lt - m)
    s = jnp.sum(e, axis=0, keepdims=True)
    scores = e / s

    iota = jax.lax.broadcasted_iota(jnp.int32, scores.shape, 0)
    cur = scores
    vals = []
    ids = []
    for _ in range(TOPK):
        v = jnp.max(cur, axis=0, keepdims=True)
        hit = cur == v
        idx = jnp.min(jnp.where(hit, iota, EXPERTS), axis=0,
                      keepdims=True)
        vals.append(v)
        ids.append(idx)
        cur = jnp.where(iota == idx, -1.0, cur)
    vals8 = jnp.concatenate(vals, axis=0)
    ids8 = jnp.concatenate(ids, axis=0)
    denom = jnp.sum(vals8, axis=0, keepdims=True) + 1e-20
    idx_ref[...] = ids8.T
    w_ref[...] = (vals8 / denom).T

    sel = (cur < 0.0).astype(jnp.float32)
    counts = jnp.sum(sel, axis=1, keepdims=True)
    sums = jnp.sum(scores, axis=1, keepdims=True)
    b = step // (seq_len // BLOCK_R)
    bio = jax.lax.broadcasted_iota(jnp.int32, (EXPERTS, bsz), 1)
    onehot = (bio == b).astype(jnp.float32)
    cnt_ref[...] += onehot * counts
    ssum_ref[...] += onehot * sums

    @pl.when(step == nsteps - 1)
    def _fin():
        ce = cnt_ref[...] * (EXPERTS / (seq_len * TOPK))
        mean_s = ssum_ref[...] * (1.0 / seq_len)
        aux_ref[...] = jnp.sum(ce * mean_s, axis=(0, 1),
                               keepdims=True) * (ALPHA / bsz)


def kernel(hidden_states, weight):
    bsz, seq_len, h = hidden_states.shape
    hs = hidden_states.reshape(bsz * seq_len, h)
    n = bsz * seq_len
    grid = n // BLOCK_R

    body = functools.partial(_gate_kernel, seq_len, bsz)
    idx, w, aux = pl.pallas_call(
        body,
        grid=(grid,),
        in_specs=[
            pl.BlockSpec((EXPERTS, h), lambda i: (0, 0)),
            pl.BlockSpec((BLOCK_R, h), lambda i: (i, 0)),
        ],
        out_specs=[
            pl.BlockSpec((BLOCK_R, TOPK), lambda i: (i, 0)),
            pl.BlockSpec((BLOCK_R, TOPK), lambda i: (i, 0)),
            pl.BlockSpec((1, 1), lambda i: (0, 0)),
        ],
        out_shape=[
            jax.ShapeDtypeStruct((n, TOPK), jnp.int32),
            jax.ShapeDtypeStruct((n, TOPK), jnp.float32),
            jax.ShapeDtypeStruct((1, 1), jnp.float32),
        ],
        scratch_shapes=[
            pltpu.VMEM((EXPERTS, bsz), jnp.float32),
            pltpu.VMEM((EXPERTS, bsz), jnp.float32),
        ],
        compiler_params=pltpu.CompilerParams(
            dimension_semantics=("arbitrary",)),
    )(weight, hs)
    return idx, w, aux[0, 0]
